# Initial kernel scaffold; baseline (speedup 1.0000x reference)
import jax
import jax.numpy as jnp
from jax.experimental import pallas as pl


def kernel(feat, tgt_lang_toks, W, b):
    raise NotImplementedError("write your pallas kernel here")


# speedup vs baseline: 5.1632x; 5.1632x over previous
import jax
import jax.numpy as jnp
from jax.experimental import pallas as pl
from jax.experimental.pallas import tpu as pltpu

S_BLK = 64


def _proj_kernel(tok_ref, feat_ref, w_ref, b_ref, out_ref):
    nb = feat_ref.shape[1]
    for j in range(nb):
        tok_j = tok_ref[j]
        x = feat_ref[:, j, :]
        w = w_ref[tok_j]
        acc = jax.lax.dot_general(
            x, w,
            dimension_numbers=(((1,), (1,)), ((), ())),
            preferred_element_type=jnp.float32,
        )
        out_ref[:, j, :] = acc + b_ref[tok_j][None, :]


def kernel(feat, tgt_lang_toks, W, b):
    S, B, C = feat.shape
    E, E_dim, _ = W.shape
    toks = tgt_lang_toks.astype(jnp.int32)

    grid_spec = pltpu.PrefetchScalarGridSpec(
        num_scalar_prefetch=1,
        grid=(S // S_BLK,),
        in_specs=[
            pl.BlockSpec((S_BLK, B, C), lambda s, tok: (s, 0, 0)),
            pl.BlockSpec((E, E_dim, C), lambda s, tok: (0, 0, 0)),
            pl.BlockSpec((E, E_dim), lambda s, tok: (0, 0)),
        ],
        out_specs=pl.BlockSpec((S_BLK, B, E_dim), lambda s, tok: (s, 0, 0)),
    )

    return pl.pallas_call(
        _proj_kernel,
        grid_spec=grid_spec,
        out_shape=jax.ShapeDtypeStruct((S, B, E_dim), feat.dtype),
    )(toks, feat, W, b)
